# baseline (speedup 1.0000x reference)
import jax
import jax.numpy as jnp
from jax.experimental import pallas as pl
from jax.experimental.pallas import tpu as pltpu

H = 128
W = 128
N = H * W
C = 2048
NCH = N // C
CH = C + 2 * W
HIDDEN = 64
HID2 = 2 * HIDDEN
NUM_COLORS = 10
FEAT_PACK = 16
FEAT2 = 2 * FEAT_PACK
LAYERS = 4
EPS = 1e-5
SENTINEL = 127

RS3 = 3.0 ** -0.5
RS4 = 0.5
RS5 = 5.0 ** -0.5


DI_FIRST = 0
DI_INT = CH
DI_LAST = 2 * CH
HL_OFF = 3 * CH
HR_OFF = 3 * CH + C
CONST_ROWS = 3 * CH + 2 * C


def _encoder_kernel(grids_ref, w_pack_ref, gcn_w_ref, gcn_b_ref,
                    ln_g_ref, ln_b_ref, w_out_ref, b_out_ref, ones_ref,
                    out_ref, x_a, x_b, const_buf):
    zero = jnp.float32(0.0)
    one = jnp.float32(1.0)

    k = jax.lax.broadcasted_iota(jnp.int32, (CH, HID2), 0)
    cw = k % W
    has_l = jnp.where(cw > 0, one, zero)
    has_r = jnp.where(cw < W - 1, one, zero)
    p = has_l * has_r
    d_int = RS4 + (RS5 - RS4) * p
    ddelta = (RS3 - RS4) + ((RS4 - RS3) - (RS5 - RS4)) * p
    me_first = jnp.where((k >= W) & (k < 2 * W), one, zero)
    me_last = jnp.where((k >= C) & (k < C + W), one, zero)
    const_buf[DI_FIRST:DI_FIRST + CH, :] = d_int + ddelta * me_first
    const_buf[DI_INT:DI_INT + CH, :] = d_int
    const_buf[DI_LAST:DI_LAST + CH, :] = d_int + ddelta * me_last
    const_buf[HL_OFF:HL_OFF + C, :] = has_l[0:C]
    const_buf[HR_OFF:HR_OFF + C, :] = has_r[0:C]

    lane = jax.lax.broadcasted_iota(jnp.int32, (CH, FEAT2), 1)
    l16 = lane % FEAT_PACK
    idx16 = jax.lax.broadcasted_iota(jnp.int32, (CH, FEAT2), 0)
    cn16 = (idx16 % W).astype(jnp.float32) * (1.0 / (W - 1))
    rbase16 = (idx16 // W).astype(jnp.float32) * (1.0 / (H - 1))

    zrow = jnp.zeros((W, HID2), jnp.float32)
    x_a[0:W, :] = zrow
    x_a[W + N:, :] = zrow
    x_b[0:W, :] = zrow
    x_b[W + N:, :] = zrow

    def embed(i):
        s = i * C
        rn = rbase16 + (1.0 / (H - 1)) \
            * (jnp.float32(C // W) * jnp.asarray(i, jnp.float32) - 1.0)
        g2 = grids_ref[0, pl.ds(s, CH), :].astype(jnp.int32)
        gs = jnp.where(lane < FEAT_PACK, g2[:, 0:1], g2[:, 1:2])
        valid = jnp.where(gs < FEAT_PACK, one, zero)
        feat = jnp.where(gs == l16, 1.0, 0.0)
        feat = jnp.where(l16 == NUM_COLORS, rn, feat)
        feat = jnp.where(l16 == NUM_COLORS + 1, cn16, feat)
        feat = jnp.where(l16 == NUM_COLORS + 2, 1.0, feat)
        feat = feat * valid
        x = jnp.dot(feat, w_pack_ref[...], preferred_element_type=jnp.float32)
        return jnp.maximum(x, zero)

    for l in range(LAYERS):
        src = None if l == 0 else (x_a, x_b)[(l + 1) % 2]
        dst = None if l == LAYERS - 1 else (x_a, x_b)[l % 2]

        def chunk_work(i, di_off, l=l, src=src, dst=dst):
            s = i * C
            dinv = const_buf[di_off:di_off + CH, :]
            if l == 0:
                xh = embed(i)
            else:
                xh = src[pl.ds(s, CH), :]
            zh = jnp.dot(xh, gcn_w_ref[l],
                         preferred_element_type=jnp.float32) * dinv
            agg = zh[W:W + C] + zh[0:C] + zh[2 * W:2 * W + C]
            agg = agg + const_buf[HL_OFF:HL_OFF + C, :] * zh[W - 1:W - 1 + C]
            agg = agg + const_buf[HR_OFF:HR_OFF + C, :] * zh[W + 1:W + 1 + C]
            d = agg * dinv[W:W + C] + gcn_b_ref[l][None, :]
            var = jnp.dot(d * d, ones_ref[...],
                          preferred_element_type=jnp.float32)
            y = d * jax.lax.rsqrt(var + EPS) * ln_g_ref[l][None, :] \
                + ln_b_ref[l][None, :]
            xn = jnp.maximum(y, zero) + xh[W:W + C]
            if l == LAYERS - 1:
                out_ref[0, pl.ds(s, C), :] = \
                    jnp.dot(xn, w_out_ref[...],
                            preferred_element_type=jnp.float32) \
                    + b_out_ref[...][None, :]
            else:
                dst[pl.ds(W + s, C), :] = xn

        chunk_work(0, DI_FIRST)

        def mid_body(i, _, work=chunk_work):
            work(i, DI_INT)
            return 0

        jax.lax.fori_loop(1, NCH - 1, mid_body, 0)
        chunk_work(NCH - 1, DI_LAST)


def _blockdiag2(m):
    r, c = m.shape[-2], m.shape[-1]
    z = jnp.zeros(m.shape[:-2] + (2 * r, 2 * c), m.dtype)
    return z.at[..., :r, :c].set(m).at[..., r:, c:].set(m)


def kernel(grids, W_in, b_in, gcn_W, gcn_b, ln_g, ln_b, W_out, b_out):
    B = grids.shape[0]
    B2 = B // 2
    feat = W_out.shape[1]
    grids2 = grids.astype(jnp.int8).reshape(B2, 2, N).transpose(0, 2, 1)
    pad = jnp.full((B2, W, 2), SENTINEL, jnp.int8)
    grids2 = jnp.concatenate([pad, grids2, pad], axis=1)

    ctr = jnp.eye(HIDDEN, dtype=jnp.float32) - 1.0 / HIDDEN
    gcn_Wc = jnp.matmul(gcn_W, ctr)
    gcn_bc = jnp.matmul(gcn_b, ctr)

    w_pack1 = jnp.concatenate(
        [W_in, b_in[None, :],
         jnp.zeros((FEAT_PACK - W_in.shape[0] - 1, HIDDEN), jnp.float32)],
        axis=0)
    w_pack = _blockdiag2(w_pack1)
    gcn_W2 = _blockdiag2(gcn_Wc)
    w_out2 = _blockdiag2(W_out)
    ones_blk = _blockdiag2(jnp.full((HIDDEN, HIDDEN), 1.0 / HIDDEN,
                                    jnp.float32))
    dup = lambda v: jnp.concatenate([v, v], axis=-1)
    gcn_b2 = dup(gcn_bc)
    ln_g2 = dup(ln_g)
    ln_b2 = dup(ln_b)
    b_out2 = dup(b_out)

    full = lambda *shape: pl.BlockSpec(shape, lambda b: (0,) * len(shape))
    out = pl.pallas_call(
        _encoder_kernel,
        grid=(B2,),
        in_specs=[
            pl.BlockSpec((1, N + 2 * W, 2), lambda b: (b, 0, 0)),
            full(*w_pack.shape),
            full(*gcn_W2.shape),
            full(*gcn_b2.shape),
            full(*ln_g2.shape),
            full(*ln_b2.shape),
            full(*w_out2.shape),
            full(*b_out2.shape),
            full(*ones_blk.shape),
        ],
        out_specs=pl.BlockSpec((1, N, HID2), lambda b: (b, 0, 0)),
        out_shape=jax.ShapeDtypeStruct((B2, N, HID2), jnp.float32),
        scratch_shapes=[
            pltpu.VMEM((N + 2 * W, HID2), jnp.float32),
            pltpu.VMEM((N + 2 * W, HID2), jnp.float32),
            pltpu.VMEM((CONST_ROWS, HID2), jnp.float32),
        ],
        compiler_params=pltpu.CompilerParams(
            dimension_semantics=("parallel",)),
    )(grids2, w_pack, gcn_W2, gcn_b2, ln_g2, ln_b2, w_out2, b_out2, ones_blk)
    return out.reshape(B2, N, 2, feat).transpose(0, 2, 1, 3).reshape(B, N, feat)


# speedup vs baseline: 1.0216x; 1.0216x over previous
import jax
import jax.numpy as jnp
from jax.experimental import pallas as pl
from jax.experimental.pallas import tpu as pltpu

H = 128
W = 128
N = H * W
C = 2048
NCH = N // C
CH = C + 2 * W
HIDDEN = 64
HID2 = 2 * HIDDEN
NUM_COLORS = 10
FEAT_PACK = 16
FEAT2 = 2 * FEAT_PACK
LAYERS = 4
EPS = 1e-5
SENTINEL = 127

RS3 = 3.0 ** -0.5
RS4 = 0.5
RS5 = 5.0 ** -0.5


def _encoder_kernel(grids_ref, w_pack_ref, gcn_w_ref, gcn_b_ref,
                    ln_g_ref, ln_b_ref, w_out_ref, b_out_ref, ones_ref,
                    out_ref, x_a, x_b):
    zero = jnp.float32(0.0)
    one = jnp.float32(1.0)

    k = jax.lax.broadcasted_iota(jnp.int32, (CH, HID2), 0)
    cw = k % W
    has_l = jnp.where(cw > 0, one, zero)
    has_r = jnp.where(cw < W - 1, one, zero)
    p = has_l * has_r
    d_int = RS4 + (RS5 - RS4) * p
    ddelta = (RS3 - RS4) + ((RS4 - RS3) - (RS5 - RS4)) * p
    me_first = jnp.where((k >= W) & (k < 2 * W), one, zero)
    me_last = jnp.where((k >= C) & (k < C + W), one, zero)
    hl_c = has_l[0:C]
    hr_c = has_r[0:C]

    lane = jax.lax.broadcasted_iota(jnp.int32, (CH, FEAT2), 1)
    l16 = lane % FEAT_PACK
    idx16 = jax.lax.broadcasted_iota(jnp.int32, (CH, FEAT2), 0)
    cn16 = (idx16 % W).astype(jnp.float32) * (1.0 / (W - 1))
    rbase16 = (idx16 // W).astype(jnp.float32) * (1.0 / (H - 1))

    zrow = jnp.zeros((W, HID2), jnp.float32)
    x_a[0:W, :] = zrow
    x_a[W + N:, :] = zrow
    x_b[0:W, :] = zrow
    x_b[W + N:, :] = zrow

    def embed(i):
        s = i * C
        rn = rbase16 + (1.0 / (H - 1)) \
            * (jnp.float32(C // W) * i.astype(jnp.float32) - 1.0)
        g2 = grids_ref[0, pl.ds(s, CH), :].astype(jnp.int32)
        gs = jnp.where(lane < FEAT_PACK, g2[:, 0:1], g2[:, 1:2])
        valid = jnp.where(gs < FEAT_PACK, one, zero)
        feat = jnp.where(gs == l16, 1.0, 0.0)
        feat = jnp.where(l16 == NUM_COLORS, rn, feat)
        feat = jnp.where(l16 == NUM_COLORS + 1, cn16, feat)
        feat = jnp.where(l16 == NUM_COLORS + 2, 1.0, feat)
        feat = feat * valid
        x = jnp.dot(feat, w_pack_ref[...], preferred_element_type=jnp.float32)
        return jnp.maximum(x, zero)

    for l in range(LAYERS):
        src = None if l == 0 else (x_a, x_b)[(l + 1) % 2]
        dst = None if l == LAYERS - 1 else (x_a, x_b)[l % 2]

        def layer_body(i, _, l=l, src=src, dst=dst):
            s = i * C
            f0 = jnp.where(i == 0, one, zero)
            f1 = jnp.where(i == NCH - 1, one, zero)
            dinv = d_int + ddelta * (me_first * f0 + me_last * f1)
            if l == 0:
                xh = embed(i)
            else:
                xh = src[pl.ds(s, CH), :]
            zh = jnp.dot(xh, gcn_w_ref[l],
                         preferred_element_type=jnp.float32) * dinv
            agg = zh[W:W + C] + zh[0:C] + zh[2 * W:2 * W + C]
            agg = agg + hl_c * zh[W - 1:W - 1 + C]
            agg = agg + hr_c * zh[W + 1:W + 1 + C]
            d = agg * dinv[W:W + C] + gcn_b_ref[l][None, :]
            var = jnp.dot(d * d, ones_ref[...],
                          preferred_element_type=jnp.float32)
            y = d * (jax.lax.rsqrt(var + EPS) * ln_g_ref[l][None, :]) \
                + ln_b_ref[l][None, :]
            xn = jnp.maximum(y, zero) + xh[W:W + C]
            if l == LAYERS - 1:
                out_ref[0, pl.ds(s, C), :] = \
                    jnp.dot(xn, w_out_ref[...],
                            preferred_element_type=jnp.float32) \
                    + b_out_ref[...][None, :]
            else:
                dst[pl.ds(W + s, C), :] = xn
            return 0

        jax.lax.fori_loop(0, NCH, layer_body, 0)


def _blockdiag2(m):
    r, c = m.shape[-2], m.shape[-1]
    z = jnp.zeros(m.shape[:-2] + (2 * r, 2 * c), m.dtype)
    return z.at[..., :r, :c].set(m).at[..., r:, c:].set(m)


def kernel(grids, W_in, b_in, gcn_W, gcn_b, ln_g, ln_b, W_out, b_out):
    B = grids.shape[0]
    B2 = B // 2
    feat = W_out.shape[1]
    grids2 = grids.astype(jnp.int8).reshape(B2, 2, N).transpose(0, 2, 1)
    pad = jnp.full((B2, W, 2), SENTINEL, jnp.int8)
    grids2 = jnp.concatenate([pad, grids2, pad], axis=1)

    ctr = jnp.eye(HIDDEN, dtype=jnp.float32) - 1.0 / HIDDEN
    gcn_Wc = jnp.matmul(gcn_W, ctr)
    gcn_bc = jnp.matmul(gcn_b, ctr)

    w_pack1 = jnp.concatenate(
        [W_in, b_in[None, :],
         jnp.zeros((FEAT_PACK - W_in.shape[0] - 1, HIDDEN), jnp.float32)],
        axis=0)
    w_pack = _blockdiag2(w_pack1)
    gcn_W2 = _blockdiag2(gcn_Wc)
    w_out2 = _blockdiag2(W_out)
    ones_blk = _blockdiag2(jnp.full((HIDDEN, HIDDEN), 1.0 / HIDDEN,
                                    jnp.float32))
    dup = lambda v: jnp.concatenate([v, v], axis=-1)
    gcn_b2 = dup(gcn_bc)
    ln_g2 = dup(ln_g)
    ln_b2 = dup(ln_b)
    b_out2 = dup(b_out)

    full = lambda *shape: pl.BlockSpec(shape, lambda b: (0,) * len(shape))
    out = pl.pallas_call(
        _encoder_kernel,
        grid=(B2,),
        in_specs=[
            pl.BlockSpec((1, N + 2 * W, 2), lambda b: (b, 0, 0)),
            full(*w_pack.shape),
            full(*gcn_W2.shape),
            full(*gcn_b2.shape),
            full(*ln_g2.shape),
            full(*ln_b2.shape),
            full(*w_out2.shape),
            full(*b_out2.shape),
            full(*ones_blk.shape),
        ],
        out_specs=pl.BlockSpec((1, N, HID2), lambda b: (b, 0, 0)),
        out_shape=jax.ShapeDtypeStruct((B2, N, HID2), jnp.float32),
        scratch_shapes=[
            pltpu.VMEM((N + 2 * W, HID2), jnp.float32),
            pltpu.VMEM((N + 2 * W, HID2), jnp.float32),
        ],
        compiler_params=pltpu.CompilerParams(
            dimension_semantics=("parallel",)),
    )(grids2, w_pack, gcn_W2, gcn_b2, ln_g2, ln_b2, w_out2, b_out2, ones_blk)
    return out.reshape(B2, N, 2, feat).transpose(0, 2, 1, 3).reshape(B, N, feat)


# speedup vs baseline: 1.0617x; 1.0392x over previous
import jax
import jax.numpy as jnp
from jax.experimental import pallas as pl
from jax.experimental.pallas import tpu as pltpu

H = 128
W = 128
N = H * W
C = 4096
NCH = N // C
CH = C + 2 * W
HIDDEN = 64
HID2 = 2 * HIDDEN
NUM_COLORS = 10
FEAT_PACK = 16
FEAT2 = 2 * FEAT_PACK
LAYERS = 4
EPS = 1e-5
SENTINEL = 127

RS3 = 3.0 ** -0.5
RS4 = 0.5
RS5 = 5.0 ** -0.5


def _encoder_kernel(grids_ref, w_pack_ref, gcn_w_ref, gcn_b_ref,
                    ln_g_ref, ln_b_ref, w_out_ref, b_out_ref, ones_ref,
                    out_ref, x_a, x_b):
    zero = jnp.float32(0.0)
    one = jnp.float32(1.0)

    k = jax.lax.broadcasted_iota(jnp.int32, (CH, HID2), 0)
    cw = k % W
    has_l = jnp.where(cw > 0, one, zero)
    has_r = jnp.where(cw < W - 1, one, zero)
    p = has_l * has_r
    d_int = RS4 + (RS5 - RS4) * p
    ddelta = (RS3 - RS4) + ((RS4 - RS3) - (RS5 - RS4)) * p
    me_first = jnp.where((k >= W) & (k < 2 * W), one, zero)
    me_last = jnp.where((k >= C) & (k < C + W), one, zero)
    hl_c = has_l[0:C]
    hr_c = has_r[0:C]

    lane = jax.lax.broadcasted_iota(jnp.int32, (CH, FEAT2), 1)
    l16 = lane % FEAT_PACK
    idx16 = jax.lax.broadcasted_iota(jnp.int32, (CH, FEAT2), 0)
    cn16 = (idx16 % W).astype(jnp.float32) * (1.0 / (W - 1))
    rbase16 = (idx16 // W).astype(jnp.float32) * (1.0 / (H - 1))

    zrow = jnp.zeros((W, HID2), jnp.float32)
    x_a[0:W, :] = zrow
    x_a[W + N:, :] = zrow
    x_b[0:W, :] = zrow
    x_b[W + N:, :] = zrow

    def embed(i):
        s = i * C
        rn = rbase16 + (1.0 / (H - 1)) \
            * (jnp.float32(C // W) * i.astype(jnp.float32) - 1.0)
        g2 = grids_ref[0, pl.ds(s, CH), :].astype(jnp.int32)
        gs = jnp.where(lane < FEAT_PACK, g2[:, 0:1], g2[:, 1:2])
        valid = jnp.where(gs < FEAT_PACK, one, zero)
        feat = jnp.where(gs == l16, 1.0, 0.0)
        feat = jnp.where(l16 == NUM_COLORS, rn, feat)
        feat = jnp.where(l16 == NUM_COLORS + 1, cn16, feat)
        feat = jnp.where(l16 == NUM_COLORS + 2, 1.0, feat)
        feat = feat * valid
        x = jnp.dot(feat, w_pack_ref[...], preferred_element_type=jnp.float32)
        return jnp.maximum(x, zero)

    for l in range(LAYERS):
        src = None if l == 0 else (x_a, x_b)[(l + 1) % 2]
        dst = None if l == LAYERS - 1 else (x_a, x_b)[l % 2]

        def layer_body(i, _, l=l, src=src, dst=dst):
            s = i * C
            f0 = jnp.where(i == 0, one, zero)
            f1 = jnp.where(i == NCH - 1, one, zero)
            dinv = d_int + ddelta * (me_first * f0 + me_last * f1)
            if l == 0:
                xh = embed(i)
            else:
                xh = src[pl.ds(s, CH), :]
            zh = jnp.dot(xh, gcn_w_ref[l],
                         preferred_element_type=jnp.float32) * dinv
            agg = zh[W:W + C] + zh[0:C] + zh[2 * W:2 * W + C]
            agg = agg + hl_c * zh[W - 1:W - 1 + C]
            agg = agg + hr_c * zh[W + 1:W + 1 + C]
            d = agg * dinv[W:W + C] + gcn_b_ref[l][None, :]
            var = jnp.dot(d * d, ones_ref[...],
                          preferred_element_type=jnp.float32)
            y = d * (jax.lax.rsqrt(var + EPS) * ln_g_ref[l][None, :]) \
                + ln_b_ref[l][None, :]
            xn = jnp.maximum(y, zero) + xh[W:W + C]
            if l == LAYERS - 1:
                out_ref[0, pl.ds(s, C), :] = \
                    jnp.dot(xn, w_out_ref[...],
                            preferred_element_type=jnp.float32) \
                    + b_out_ref[...][None, :]
            else:
                dst[pl.ds(W + s, C), :] = xn
            return 0

        jax.lax.fori_loop(0, NCH, layer_body, 0)


def _blockdiag2(m):
    r, c = m.shape[-2], m.shape[-1]
    z = jnp.zeros(m.shape[:-2] + (2 * r, 2 * c), m.dtype)
    return z.at[..., :r, :c].set(m).at[..., r:, c:].set(m)


def kernel(grids, W_in, b_in, gcn_W, gcn_b, ln_g, ln_b, W_out, b_out):
    B = grids.shape[0]
    B2 = B // 2
    feat = W_out.shape[1]
    grids2 = grids.astype(jnp.int8).reshape(B2, 2, N).transpose(0, 2, 1)
    pad = jnp.full((B2, W, 2), SENTINEL, jnp.int8)
    grids2 = jnp.concatenate([pad, grids2, pad], axis=1)

    ctr = jnp.eye(HIDDEN, dtype=jnp.float32) - 1.0 / HIDDEN
    gcn_Wc = jnp.matmul(gcn_W, ctr)
    gcn_bc = jnp.matmul(gcn_b, ctr)

    w_pack1 = jnp.concatenate(
        [W_in, b_in[None, :],
         jnp.zeros((FEAT_PACK - W_in.shape[0] - 1, HIDDEN), jnp.float32)],
        axis=0)
    w_pack = _blockdiag2(w_pack1)
    gcn_W2 = _blockdiag2(gcn_Wc)
    w_out2 = _blockdiag2(W_out)
    ones_blk = _blockdiag2(jnp.full((HIDDEN, HIDDEN), 1.0 / HIDDEN,
                                    jnp.float32))
    dup = lambda v: jnp.concatenate([v, v], axis=-1)
    gcn_b2 = dup(gcn_bc)
    ln_g2 = dup(ln_g)
    ln_b2 = dup(ln_b)
    b_out2 = dup(b_out)

    full = lambda *shape: pl.BlockSpec(shape, lambda b: (0,) * len(shape))
    out = pl.pallas_call(
        _encoder_kernel,
        grid=(B2,),
        in_specs=[
            pl.BlockSpec((1, N + 2 * W, 2), lambda b: (b, 0, 0)),
            full(*w_pack.shape),
            full(*gcn_W2.shape),
            full(*gcn_b2.shape),
            full(*ln_g2.shape),
            full(*ln_b2.shape),
            full(*w_out2.shape),
            full(*b_out2.shape),
            full(*ones_blk.shape),
        ],
        out_specs=pl.BlockSpec((1, N, HID2), lambda b: (b, 0, 0)),
        out_shape=jax.ShapeDtypeStruct((B2, N, HID2), jnp.float32),
        scratch_shapes=[
            pltpu.VMEM((N + 2 * W, HID2), jnp.float32),
            pltpu.VMEM((N + 2 * W, HID2), jnp.float32),
        ],
        compiler_params=pltpu.CompilerParams(
            dimension_semantics=("parallel",)),
    )(grids2, w_pack, gcn_W2, gcn_b2, ln_g2, ln_b2, w_out2, b_out2, ones_blk)
    return out.reshape(B2, N, 2, feat).transpose(0, 2, 1, 3).reshape(B, N, feat)


# speedup vs baseline: 1.0678x; 1.0058x over previous
import jax
import jax.numpy as jnp
from jax.experimental import pallas as pl
from jax.experimental.pallas import tpu as pltpu

H = 128
W = 128
N = H * W
C = 8192
NCH = N // C
CH = C + 2 * W
HIDDEN = 64
HID2 = 2 * HIDDEN
NUM_COLORS = 10
FEAT_PACK = 16
FEAT2 = 2 * FEAT_PACK
LAYERS = 4
EPS = 1e-5
SENTINEL = 127

RS3 = 3.0 ** -0.5
RS4 = 0.5
RS5 = 5.0 ** -0.5


def _encoder_kernel(grids_ref, w_pack_ref, gcn_w_ref, gcn_b_ref,
                    ln_g_ref, ln_b_ref, w_out_ref, b_out_ref, ones_ref,
                    out_ref, x_a, x_b):
    zero = jnp.float32(0.0)
    one = jnp.float32(1.0)

    k = jax.lax.broadcasted_iota(jnp.int32, (CH, HID2), 0)
    cw = k % W
    has_l = jnp.where(cw > 0, one, zero)
    has_r = jnp.where(cw < W - 1, one, zero)
    p = has_l * has_r
    d_int = RS4 + (RS5 - RS4) * p
    ddelta = (RS3 - RS4) + ((RS4 - RS3) - (RS5 - RS4)) * p
    me_first = jnp.where((k >= W) & (k < 2 * W), one, zero)
    me_last = jnp.where((k >= C) & (k < C + W), one, zero)
    hl_c = has_l[0:C]
    hr_c = has_r[0:C]

    lane = jax.lax.broadcasted_iota(jnp.int32, (CH, FEAT2), 1)
    l16 = lane % FEAT_PACK
    idx16 = jax.lax.broadcasted_iota(jnp.int32, (CH, FEAT2), 0)
    cn16 = (idx16 % W).astype(jnp.float32) * (1.0 / (W - 1))
    rbase16 = (idx16 // W).astype(jnp.float32) * (1.0 / (H - 1))

    zrow = jnp.zeros((W, HID2), jnp.float32)
    x_a[0:W, :] = zrow
    x_a[W + N:, :] = zrow
    x_b[0:W, :] = zrow
    x_b[W + N:, :] = zrow

    def embed(i):
        s = i * C
        rn = rbase16 + (1.0 / (H - 1)) \
            * (jnp.float32(C // W) * i.astype(jnp.float32) - 1.0)
        g2 = grids_ref[0, pl.ds(s, CH), :].astype(jnp.int32)
        gs = jnp.where(lane < FEAT_PACK, g2[:, 0:1], g2[:, 1:2])
        valid = jnp.where(gs < FEAT_PACK, one, zero)
        feat = jnp.where(gs == l16, 1.0, 0.0)
        feat = jnp.where(l16 == NUM_COLORS, rn, feat)
        feat = jnp.where(l16 == NUM_COLORS + 1, cn16, feat)
        feat = jnp.where(l16 == NUM_COLORS + 2, 1.0, feat)
        feat = feat * valid
        x = jnp.dot(feat, w_pack_ref[...], preferred_element_type=jnp.float32)
        return jnp.maximum(x, zero)

    for l in range(LAYERS):
        src = None if l == 0 else (x_a, x_b)[(l + 1) % 2]
        dst = None if l == LAYERS - 1 else (x_a, x_b)[l % 2]

        def layer_body(i, _, l=l, src=src, dst=dst):
            s = i * C
            f0 = jnp.where(i == 0, one, zero)
            f1 = jnp.where(i == NCH - 1, one, zero)
            dinv = d_int + ddelta * (me_first * f0 + me_last * f1)
            if l == 0:
                xh = embed(i)
            else:
                xh = src[pl.ds(s, CH), :]
            zh = jnp.dot(xh, gcn_w_ref[l],
                         preferred_element_type=jnp.float32) * dinv
            agg = zh[W:W + C] + zh[0:C] + zh[2 * W:2 * W + C]
            agg = agg + hl_c * zh[W - 1:W - 1 + C]
            agg = agg + hr_c * zh[W + 1:W + 1 + C]
            d = agg * dinv[W:W + C] + gcn_b_ref[l][None, :]
            var = jnp.dot(d * d, ones_ref[...],
                          preferred_element_type=jnp.float32)
            y = d * (jax.lax.rsqrt(var + EPS) * ln_g_ref[l][None, :]) \
                + ln_b_ref[l][None, :]
            xn = jnp.maximum(y, zero) + xh[W:W + C]
            if l == LAYERS - 1:
                out_ref[0, pl.ds(s, C), :] = \
                    jnp.dot(xn, w_out_ref[...],
                            preferred_element_type=jnp.float32) \
                    + b_out_ref[...][None, :]
            else:
                dst[pl.ds(W + s, C), :] = xn
            return 0

        jax.lax.fori_loop(0, NCH, layer_body, 0)


def _blockdiag2(m):
    r, c = m.shape[-2], m.shape[-1]
    z = jnp.zeros(m.shape[:-2] + (2 * r, 2 * c), m.dtype)
    return z.at[..., :r, :c].set(m).at[..., r:, c:].set(m)


def kernel(grids, W_in, b_in, gcn_W, gcn_b, ln_g, ln_b, W_out, b_out):
    B = grids.shape[0]
    B2 = B // 2
    feat = W_out.shape[1]
    grids2 = grids.astype(jnp.int8).reshape(B2, 2, N).transpose(0, 2, 1)
    pad = jnp.full((B2, W, 2), SENTINEL, jnp.int8)
    grids2 = jnp.concatenate([pad, grids2, pad], axis=1)

    ctr = jnp.eye(HIDDEN, dtype=jnp.float32) - 1.0 / HIDDEN
    gcn_Wc = jnp.matmul(gcn_W, ctr)
    gcn_bc = jnp.matmul(gcn_b, ctr)

    w_pack1 = jnp.concatenate(
        [W_in, b_in[None, :],
         jnp.zeros((FEAT_PACK - W_in.shape[0] - 1, HIDDEN), jnp.float32)],
        axis=0)
    w_pack = _blockdiag2(w_pack1)
    gcn_W2 = _blockdiag2(gcn_Wc)
    w_out2 = _blockdiag2(W_out)
    ones_blk = _blockdiag2(jnp.full((HIDDEN, HIDDEN), 1.0 / HIDDEN,
                                    jnp.float32))
    dup = lambda v: jnp.concatenate([v, v], axis=-1)
    gcn_b2 = dup(gcn_bc)
    ln_g2 = dup(ln_g)
    ln_b2 = dup(ln_b)
    b_out2 = dup(b_out)

    full = lambda *shape: pl.BlockSpec(shape, lambda b: (0,) * len(shape))
    out = pl.pallas_call(
        _encoder_kernel,
        grid=(B2,),
        in_specs=[
            pl.BlockSpec((1, N + 2 * W, 2), lambda b: (b, 0, 0)),
            full(*w_pack.shape),
            full(*gcn_W2.shape),
            full(*gcn_b2.shape),
            full(*ln_g2.shape),
            full(*ln_b2.shape),
            full(*w_out2.shape),
            full(*b_out2.shape),
            full(*ones_blk.shape),
        ],
        out_specs=pl.BlockSpec((1, N, HID2), lambda b: (b, 0, 0)),
        out_shape=jax.ShapeDtypeStruct((B2, N, HID2), jnp.float32),
        scratch_shapes=[
            pltpu.VMEM((N + 2 * W, HID2), jnp.float32),
            pltpu.VMEM((N + 2 * W, HID2), jnp.float32),
        ],
        compiler_params=pltpu.CompilerParams(
            dimension_semantics=("parallel",)),
    )(grids2, w_pack, gcn_W2, gcn_b2, ln_g2, ln_b2, w_out2, b_out2, ones_blk)
    return out.reshape(B2, N, 2, feat).transpose(0, 2, 1, 3).reshape(B, N, feat)


# speedup vs baseline: 1.5169x; 1.4205x over previous
import jax
import jax.numpy as jnp
from jax.experimental import pallas as pl
from jax.experimental.pallas import tpu as pltpu

H = 128
W = 128
N = H * W
C = 4096
NCH = N // C
CH = C + 2 * W
HIDDEN = 64
HID2 = 2 * HIDDEN
NUM_COLORS = 10
FEAT_PACK = 16
FEAT2 = 2 * FEAT_PACK
LAYERS = 4
EPS = 1e-5
SENTINEL = 127

RS3 = 3.0 ** -0.5
RS4 = 0.5
RS5 = 5.0 ** -0.5


def _encoder_kernel(grids_ref, w_pack_ref, gcn_w_ref, gcn_b_ref,
                    ln_g_ref, ln_b_ref, w_out0_ref, w_out1_ref, b_out_ref,
                    ones_ref, out_ref, x_buf):
    zero = jnp.float32(0.0)
    one = jnp.float32(1.0)

    k = jax.lax.broadcasted_iota(jnp.int32, (CH, HID2), 0)
    cw = k % W
    has_l = jnp.where(cw > 0, one, zero)
    has_r = jnp.where(cw < W - 1, one, zero)
    p = has_l * has_r
    d_int = RS4 + (RS5 - RS4) * p
    ddelta = (RS3 - RS4) + ((RS4 - RS3) - (RS5 - RS4)) * p
    me_first = jnp.where((k >= W) & (k < 2 * W), one, zero)
    me_last = jnp.where((k >= C) & (k < C + W), one, zero)
    hl_c = has_l[0:C]
    hr_c = has_r[0:C]

    lane = jax.lax.broadcasted_iota(jnp.int32, (CH, FEAT2), 1)
    l16 = lane % FEAT_PACK
    idx16 = jax.lax.broadcasted_iota(jnp.int32, (CH, FEAT2), 0)
    cn16 = (idx16 % W).astype(jnp.float32) * (1.0 / (W - 1))
    rbase16 = (idx16 // W).astype(jnp.float32) * (1.0 / (H - 1))

    zrow = jnp.zeros((W, HID2), jnp.float32)
    x_buf[0:W, :] = zrow
    x_buf[W + N:, :] = zrow

    def embed(i):
        s = i * C
        rn = rbase16 + (1.0 / (H - 1)) \
            * (jnp.float32(C // W) * i.astype(jnp.float32) - 1.0)
        g2 = grids_ref[0, pl.ds(s, CH), :].astype(jnp.int32)
        gs = jnp.where(lane < FEAT_PACK, g2[:, 0:1], g2[:, 1:2])
        valid = jnp.where(gs < FEAT_PACK, one, zero)
        feat = jnp.where(gs == l16, 1.0, 0.0)
        feat = jnp.where(l16 == NUM_COLORS, rn, feat)
        feat = jnp.where(l16 == NUM_COLORS + 1, cn16, feat)
        feat = jnp.where(l16 == NUM_COLORS + 2, 1.0, feat)
        feat = feat * valid
        x = jnp.dot(feat, w_pack_ref[...], preferred_element_type=jnp.float32)
        return jnp.maximum(x, zero)

    for l in range(LAYERS):

        def layer_body(i, carry, l=l):
            s = i * C
            f0 = jnp.where(i == 0, one, zero)
            f1 = jnp.where(i == NCH - 1, one, zero)
            dinv = d_int + ddelta * (me_first * f0 + me_last * f1)
            if l == 0:
                xh = embed(i)
                new_carry = carry
            else:
                rest = x_buf[pl.ds(s + W, C + W), :]
                new_carry = rest[C - W:C]
                xh = jnp.concatenate([carry, rest], axis=0)
            zh = jnp.dot(xh, gcn_w_ref[l],
                         preferred_element_type=jnp.float32) * dinv
            agg = zh[W:W + C] + zh[0:C] + zh[2 * W:2 * W + C]
            agg = agg + hl_c * zh[W - 1:W - 1 + C]
            agg = agg + hr_c * zh[W + 1:W + 1 + C]
            d = agg * dinv[W:W + C] + gcn_b_ref[l][None, :]
            var = jnp.dot(d * d, ones_ref[...],
                          preferred_element_type=jnp.float32)
            y = d * (jax.lax.rsqrt(var + EPS) * ln_g_ref[l][None, :]) \
                + ln_b_ref[l][None, :]
            xn = jnp.maximum(y, zero) + xh[W:W + C]
            if l == LAYERS - 1:
                out_ref[0, pl.ds(s, C), :] = \
                    jnp.dot(xn, w_out0_ref[...],
                            preferred_element_type=jnp.float32) \
                    + b_out_ref[...][None, :]
                out_ref[1, pl.ds(s, C), :] = \
                    jnp.dot(xn, w_out1_ref[...],
                            preferred_element_type=jnp.float32) \
                    + b_out_ref[...][None, :]
            else:
                x_buf[pl.ds(W + s, C), :] = xn
            return new_carry

        jax.lax.fori_loop(0, NCH, layer_body, zrow)


def _blockdiag2(m):
    r, c = m.shape[-2], m.shape[-1]
    z = jnp.zeros(m.shape[:-2] + (2 * r, 2 * c), m.dtype)
    return z.at[..., :r, :c].set(m).at[..., r:, c:].set(m)


def kernel(grids, W_in, b_in, gcn_W, gcn_b, ln_g, ln_b, W_out, b_out):
    B = grids.shape[0]
    B2 = B // 2
    feat = W_out.shape[1]
    grids2 = grids.astype(jnp.int8).reshape(B2, 2, N).transpose(0, 2, 1)
    pad = jnp.full((B2, W, 2), SENTINEL, jnp.int8)
    grids2 = jnp.concatenate([pad, grids2, pad], axis=1)

    ctr = jnp.eye(HIDDEN, dtype=jnp.float32) - 1.0 / HIDDEN
    gcn_Wc = jnp.matmul(gcn_W, ctr)
    gcn_bc = jnp.matmul(gcn_b, ctr)

    w_pack1 = jnp.concatenate(
        [W_in, b_in[None, :],
         jnp.zeros((FEAT_PACK - W_in.shape[0] - 1, HIDDEN), jnp.float32)],
        axis=0)
    w_pack = _blockdiag2(w_pack1)
    gcn_W2 = _blockdiag2(gcn_Wc)
    zout = jnp.zeros_like(W_out)
    w_out0 = jnp.concatenate([W_out, zout], axis=0)
    w_out1 = jnp.concatenate([zout, W_out], axis=0)
    ones_blk = _blockdiag2(jnp.full((HIDDEN, HIDDEN), 1.0 / HIDDEN,
                                    jnp.float32))
    dup = lambda v: jnp.concatenate([v, v], axis=-1)
    gcn_b2 = dup(gcn_bc)
    ln_g2 = dup(ln_g)
    ln_b2 = dup(ln_b)

    full = lambda *shape: pl.BlockSpec(shape, lambda b: (0,) * len(shape))
    out = pl.pallas_call(
        _encoder_kernel,
        grid=(B2,),
        in_specs=[
            pl.BlockSpec((1, N + 2 * W, 2), lambda b: (b, 0, 0)),
            full(*w_pack.shape),
            full(*gcn_W2.shape),
            full(*gcn_b2.shape),
            full(*ln_g2.shape),
            full(*ln_b2.shape),
            full(*w_out0.shape),
            full(*w_out1.shape),
            full(*b_out.shape),
            full(*ones_blk.shape),
        ],
        out_specs=pl.BlockSpec((2, N, feat), lambda b: (b, 0, 0)),
        out_shape=jax.ShapeDtypeStruct((B, N, feat), jnp.float32),
        scratch_shapes=[
            pltpu.VMEM((N + 2 * W, HID2), jnp.float32),
        ],
        compiler_params=pltpu.CompilerParams(
            dimension_semantics=("parallel",)),
    )(grids2, w_pack, gcn_W2, gcn_b2, ln_g2, ln_b2, w_out0, w_out1, b_out,
      ones_blk)
    return out
